# baseline (speedup 1.0000x reference)
import functools

import jax
import jax.numpy as jnp
from jax import lax
from jax.experimental import pallas as pl
from jax.experimental.pallas import tpu as pltpu
from jax.experimental.pallas import tpu_sc as plsc

N_ROWS = 64
N_COLS = 32768
NUM_CORES = 2
NUM_SUBCORES = 16
NUM_WORKERS = NUM_CORES * NUM_SUBCORES
SC_ROWS = NUM_WORKERS
TC_ROWS = N_ROWS - SC_ROWS
LANES = 16
UNROLL = 8
ITERS_PER_ROW = N_COLS // (UNROLL * LANES)
TC_BLOCK = 2048
TC_STEPS = N_COLS // TC_BLOCK

_mesh = plsc.VectorSubcoreMesh(core_axis_name="c", subcore_axis_name="s")


def _lex_min(av, ai, bv, bi):
    upd = (bv < av) | ((bv == av) & (bi < ai))
    return jnp.where(upd, bv, av), jnp.where(upd, bi, ai)


@functools.partial(
    pl.kernel,
    mesh=_mesh,
    out_type=jax.ShapeDtypeStruct((NUM_WORKERS, LANES), jnp.int32),
    scratch_types=[
        pltpu.VMEM((N_COLS,), jnp.float32),
        pltpu.VMEM((LANES,), jnp.int32),
        pltpu.SemaphoreType.DMA,
    ],
)
def _argmin_sc(x_hbm, out_hbm, buf, res_v, sem):
    wid = lax.axis_index("s") * NUM_CORES + lax.axis_index("c")
    pltpu.async_copy(x_hbm.at[wid], buf, sem).wait()

    lane_iota = lax.iota(jnp.int32, LANES)

    def row_body(it, carry):
        acc = list(carry)
        git_vec = jnp.full((LANES,), it, jnp.int32)
        off = it * (UNROLL * LANES)
        for u in range(UNROLL):
            bv, bi = acc[2 * u], acc[2 * u + 1]
            v = buf[pl.ds(off + u * LANES, LANES)]
            upd = v < bv
            acc[2 * u] = jnp.where(upd, v, bv)
            acc[2 * u + 1] = jnp.where(upd, git_vec, bi)
        return tuple(acc)

    init = []
    for _ in range(UNROLL):
        init += [jnp.full((LANES,), jnp.inf, jnp.float32),
                 jnp.zeros((LANES,), jnp.int32)]
    carry = lax.fori_loop(0, ITERS_PER_ROW, row_body, tuple(init))

    mv, mi = None, None
    for u in range(UNROLL):
        gidx = carry[2 * u + 1] * (UNROLL * LANES) + (lane_iota + u * LANES)
        if mv is None:
            mv, mi = carry[2 * u], gidx
        else:
            mv, mi = _lex_min(mv, mi, carry[2 * u], gidx)

    for shift in (8, 4, 2, 1):
        partner = lane_iota ^ shift
        pv = mv.at[partner].get(mode="promise_in_bounds", unique_indices=True)
        pi = mi.at[partner].get(mode="promise_in_bounds", unique_indices=True)
        mv, mi = _lex_min(mv, mi, pv, pi)

    res_v[...] = mi
    pltpu.sync_copy(res_v, out_hbm.at[wid])


def _argmin_tc_body(x_ref, out_ref, run_min, run_arg):
    i = pl.program_id(0)
    xb = x_ref[...]
    bmin = jnp.min(xb, axis=1, keepdims=True)
    iota2 = lax.broadcasted_iota(jnp.int32, (TC_ROWS, TC_BLOCK), 1)
    masked = jnp.where(xb == bmin, iota2, jnp.int32(2**30))
    barg = jnp.min(masked, axis=1, keepdims=True) + i * TC_BLOCK

    @pl.when(i == 0)
    def _():
        run_min[...] = bmin
        run_arg[...] = barg

    @pl.when(i > 0)
    def _():
        prev_min = run_min[...]
        prev_arg = run_arg[...]
        upd = bmin < prev_min
        run_min[...] = jnp.where(upd, bmin, prev_min)
        run_arg[...] = jnp.where(upd, barg, prev_arg)

    @pl.when(i == TC_STEPS - 1)
    def _():
        out_ref[...] = run_arg[...]


_argmin_tc = pl.pallas_call(
    _argmin_tc_body,
    grid=(TC_STEPS,),
    in_specs=[pl.BlockSpec((TC_ROWS, TC_BLOCK), lambda i: (1, i))],
    out_specs=pl.BlockSpec((TC_ROWS, 1), lambda i: (0, 0)),
    out_shape=jax.ShapeDtypeStruct((TC_ROWS, 1), jnp.int32),
    scratch_shapes=[
        pltpu.VMEM((TC_ROWS, 1), jnp.float32),
        pltpu.VMEM((TC_ROWS, 1), jnp.int32),
    ],
)


def kernel(x):
    sc2d = _argmin_sc(x)
    tc2d = _argmin_tc(x)
    return jnp.concatenate([sc2d[:, 0], tc2d[:, 0]])


# speedup vs baseline: 1.0684x; 1.0684x over previous
import functools

import jax
import jax.numpy as jnp
from jax import lax
from jax.experimental import pallas as pl
from jax.experimental.pallas import tpu as pltpu
from jax.experimental.pallas import tpu_sc as plsc

N_ROWS = 64
N_COLS = 32768
NUM_CORES = 2
NUM_SUBCORES = 16
NUM_WORKERS = NUM_CORES * NUM_SUBCORES
SC_ROWS = NUM_WORKERS
TC_ROWS = N_ROWS - SC_ROWS
LANES = 16
UNROLL = 8
ITERS_PER_ROW = N_COLS // (UNROLL * LANES)
TC_BLOCK = 2048
TC_STEPS = N_COLS // TC_BLOCK

_mesh = plsc.VectorSubcoreMesh(core_axis_name="c", subcore_axis_name="s")


def _lex_min(av, ai, bv, bi):
    upd = (bv < av) | ((bv == av) & (bi < ai))
    return jnp.where(upd, bv, av), jnp.where(upd, bi, ai)


@functools.partial(
    pl.kernel,
    mesh=_mesh,
    out_type=jax.ShapeDtypeStruct((NUM_WORKERS, LANES), jnp.int32),
    scratch_types=[
        pltpu.VMEM((N_COLS,), jnp.float32),
        pltpu.VMEM((LANES,), jnp.int32),
        pltpu.SemaphoreType.DMA,
    ],
)
def _argmin_sc(x_hbm, out_hbm, buf, res_v, sem):
    wid = lax.axis_index("s") * NUM_CORES + lax.axis_index("c")
    pltpu.async_copy(x_hbm.at[wid], buf, sem).wait()

    lane_iota = lax.iota(jnp.int32, LANES)

    def row_body(it, carry):
        acc = list(carry)
        git_vec = jnp.full((LANES,), it, jnp.int32)
        off = it * (UNROLL * LANES)
        for u in range(UNROLL):
            bv, bi = acc[2 * u], acc[2 * u + 1]
            v = buf[pl.ds(off + u * LANES, LANES)]
            upd = v < bv
            acc[2 * u] = jnp.where(upd, v, bv)
            acc[2 * u + 1] = jnp.where(upd, git_vec, bi)
        return tuple(acc)

    init = []
    for _ in range(UNROLL):
        init += [jnp.full((LANES,), jnp.inf, jnp.float32),
                 jnp.zeros((LANES,), jnp.int32)]
    carry = lax.fori_loop(0, ITERS_PER_ROW, row_body, tuple(init))

    mv, mi = None, None
    for u in range(UNROLL):
        gidx = carry[2 * u + 1] * (UNROLL * LANES) + (lane_iota + u * LANES)
        if mv is None:
            mv, mi = carry[2 * u], gidx
        else:
            mv, mi = _lex_min(mv, mi, carry[2 * u], gidx)

    for shift in (8, 4, 2, 1):
        partner = lane_iota ^ shift
        pv = mv.at[partner].get(mode="promise_in_bounds", unique_indices=True)
        pi = mi.at[partner].get(mode="promise_in_bounds", unique_indices=True)
        mv, mi = _lex_min(mv, mi, pv, pi)

    res_v[...] = mi
    pltpu.sync_copy(res_v, out_hbm.at[wid])


_TC_SUB = TC_BLOCK // 128


def _argmin_tc_body(x_ref, out_ref, run_min, run_arg):
    i = pl.program_id(0)
    rm = x_ref[:, pl.ds(0, 128)]
    ra = jnp.full((TC_ROWS, 128), i * _TC_SUB, jnp.int32)
    for s in range(1, _TC_SUB):
        vs = x_ref[:, pl.ds(s * 128, 128)]
        upd = vs < rm
        rm = jnp.where(upd, vs, rm)
        ra = jnp.where(upd, jnp.int32(i * _TC_SUB + s), ra)

    @pl.when(i == 0)
    def _():
        run_min[...] = rm
        run_arg[...] = ra

    @pl.when(i > 0)
    def _():
        prev_min = run_min[...]
        prev_arg = run_arg[...]
        upd = rm < prev_min
        run_min[...] = jnp.where(upd, rm, prev_min)
        run_arg[...] = jnp.where(upd, ra, prev_arg)

    @pl.when(i == TC_STEPS - 1)
    def _():
        gmin = run_min[...]
        col = run_arg[...] * 128 + lax.broadcasted_iota(
            jnp.int32, (TC_ROWS, 128), 1)
        m = jnp.min(gmin, axis=1, keepdims=True)
        out_ref[...] = jnp.min(
            jnp.where(gmin == m, col, jnp.int32(2**30)),
            axis=1, keepdims=True)


_argmin_tc = pl.pallas_call(
    _argmin_tc_body,
    grid=(TC_STEPS,),
    in_specs=[pl.BlockSpec((TC_ROWS, TC_BLOCK), lambda i: (1, i))],
    out_specs=pl.BlockSpec((TC_ROWS, 1), lambda i: (0, 0)),
    out_shape=jax.ShapeDtypeStruct((TC_ROWS, 1), jnp.int32),
    scratch_shapes=[
        pltpu.VMEM((TC_ROWS, 128), jnp.float32),
        pltpu.VMEM((TC_ROWS, 128), jnp.int32),
    ],
)


def kernel(x):
    sc2d = _argmin_sc(x)
    tc2d = _argmin_tc(x)
    return jnp.concatenate([sc2d[:, 0], tc2d[:, 0]])


# speedup vs baseline: 1.1980x; 1.1212x over previous
import functools

import jax
import jax.numpy as jnp
from jax import lax
from jax.experimental import pallas as pl
from jax.experimental.pallas import tpu as pltpu
from jax.experimental.pallas import tpu_sc as plsc

N_ROWS = 64
N_COLS = 32768
NUM_CORES = 2
NUM_SUBCORES = 16
NUM_WORKERS = NUM_CORES * NUM_SUBCORES
SC_ROWS = NUM_WORKERS
TC_ROWS = N_ROWS - SC_ROWS
LANES = 16
UNROLL = 8
ITERS_PER_ROW = N_COLS // (UNROLL * LANES)
TC_BLOCK = 4096
TC_STEPS = N_COLS // TC_BLOCK

_mesh = plsc.VectorSubcoreMesh(core_axis_name="c", subcore_axis_name="s")


def _lex_min(av, ai, bv, bi):
    upd = (bv < av) | ((bv == av) & (bi < ai))
    return jnp.where(upd, bv, av), jnp.where(upd, bi, ai)


@functools.partial(
    pl.kernel,
    mesh=_mesh,
    out_type=jax.ShapeDtypeStruct((NUM_WORKERS, LANES), jnp.int32),
    scratch_types=[
        pltpu.VMEM((N_COLS,), jnp.float32),
        pltpu.VMEM((LANES,), jnp.int32),
        pltpu.SemaphoreType.DMA,
    ],
)
def _argmin_sc(x_hbm, out_hbm, buf, res_v, sem):
    wid = lax.axis_index("s") * NUM_CORES + lax.axis_index("c")
    pltpu.async_copy(x_hbm.at[wid], buf, sem).wait()

    lane_iota = lax.iota(jnp.int32, LANES)

    def row_body(it, carry):
        acc = list(carry)
        git_vec = jnp.full((LANES,), it, jnp.int32)
        off = it * (UNROLL * LANES)
        for u in range(UNROLL):
            bv, bi = acc[2 * u], acc[2 * u + 1]
            v = buf[pl.ds(off + u * LANES, LANES)]
            upd = v < bv
            acc[2 * u] = jnp.where(upd, v, bv)
            acc[2 * u + 1] = jnp.where(upd, git_vec, bi)
        return tuple(acc)

    init = []
    for _ in range(UNROLL):
        init += [jnp.full((LANES,), jnp.inf, jnp.float32),
                 jnp.zeros((LANES,), jnp.int32)]
    carry = lax.fori_loop(0, ITERS_PER_ROW, row_body, tuple(init))

    mv, mi = None, None
    for u in range(UNROLL):
        gidx = carry[2 * u + 1] * (UNROLL * LANES) + (lane_iota + u * LANES)
        if mv is None:
            mv, mi = carry[2 * u], gidx
        else:
            mv, mi = _lex_min(mv, mi, carry[2 * u], gidx)

    for shift in (8, 4, 2, 1):
        partner = lane_iota ^ shift
        pv = mv.at[partner].get(mode="promise_in_bounds", unique_indices=True)
        pi = mi.at[partner].get(mode="promise_in_bounds", unique_indices=True)
        mv, mi = _lex_min(mv, mi, pv, pi)

    res_v[...] = mi
    pltpu.sync_copy(res_v, out_hbm.at[wid])


_TC_SUB = TC_BLOCK // 128


def _argmin_tc_body(x_ref, out_ref, run_min, run_arg):
    i = pl.program_id(0)
    rm = x_ref[:, pl.ds(0, 128)]
    ra = jnp.full((TC_ROWS, 128), i * _TC_SUB, jnp.int32)
    for s in range(1, _TC_SUB):
        vs = x_ref[:, pl.ds(s * 128, 128)]
        upd = vs < rm
        rm = jnp.where(upd, vs, rm)
        ra = jnp.where(upd, jnp.int32(i * _TC_SUB + s), ra)

    @pl.when(i == 0)
    def _():
        run_min[...] = rm
        run_arg[...] = ra

    @pl.when(i > 0)
    def _():
        prev_min = run_min[...]
        prev_arg = run_arg[...]
        upd = rm < prev_min
        run_min[...] = jnp.where(upd, rm, prev_min)
        run_arg[...] = jnp.where(upd, ra, prev_arg)

    @pl.when(i == TC_STEPS - 1)
    def _():
        gmin = run_min[...]
        col = run_arg[...] * 128 + lax.broadcasted_iota(
            jnp.int32, (TC_ROWS, 128), 1)
        m = jnp.min(gmin, axis=1, keepdims=True)
        out_ref[...] = jnp.min(
            jnp.where(gmin == m, col, jnp.int32(2**30)),
            axis=1, keepdims=True)


_argmin_tc = pl.pallas_call(
    _argmin_tc_body,
    grid=(TC_STEPS,),
    in_specs=[pl.BlockSpec((TC_ROWS, TC_BLOCK), lambda i: (1, i))],
    out_specs=pl.BlockSpec((TC_ROWS, 1), lambda i: (0, 0)),
    out_shape=jax.ShapeDtypeStruct((TC_ROWS, 1), jnp.int32),
    scratch_shapes=[
        pltpu.VMEM((TC_ROWS, 128), jnp.float32),
        pltpu.VMEM((TC_ROWS, 128), jnp.int32),
    ],
)


def kernel(x):
    sc2d = _argmin_sc(x)
    tc2d = _argmin_tc(x)
    return jnp.concatenate([sc2d[:, 0], tc2d[:, 0]])


# speedup vs baseline: 3.4758x; 2.9014x over previous
import jax
import jax.numpy as jnp
from jax import lax
from jax.experimental import pallas as pl
from jax.experimental.pallas import tpu as pltpu

N_ROWS = 64
N_COLS = 32768
BLOCK = 4096
STEPS = N_COLS // BLOCK
SUB = BLOCK // 128


def _argmin_body(x_ref, out_ref, run_min, run_arg):
    i = pl.program_id(0)
    rm = x_ref[:, pl.ds(0, 128)]
    ra = jnp.full((N_ROWS, 128), i * SUB, jnp.int32)
    for s in range(1, SUB):
        vs = x_ref[:, pl.ds(s * 128, 128)]
        upd = vs < rm
        rm = jnp.where(upd, vs, rm)
        ra = jnp.where(upd, jnp.int32(i * SUB + s), ra)

    @pl.when(i == 0)
    def _():
        run_min[...] = rm
        run_arg[...] = ra

    @pl.when(i > 0)
    def _():
        prev_min = run_min[...]
        prev_arg = run_arg[...]
        upd = rm < prev_min
        run_min[...] = jnp.where(upd, rm, prev_min)
        run_arg[...] = jnp.where(upd, ra, prev_arg)

    @pl.when(i == STEPS - 1)
    def _():
        gmin = run_min[...]
        col = run_arg[...] * 128 + lax.broadcasted_iota(
            jnp.int32, (N_ROWS, 128), 1)
        m = jnp.min(gmin, axis=1, keepdims=True)
        out_ref[...] = jnp.min(
            jnp.where(gmin == m, col, jnp.int32(2**30)),
            axis=1, keepdims=True)


_argmin = pl.pallas_call(
    _argmin_body,
    grid=(STEPS,),
    in_specs=[pl.BlockSpec((N_ROWS, BLOCK), lambda i: (0, i))],
    out_specs=pl.BlockSpec((N_ROWS, 1), lambda i: (0, 0)),
    out_shape=jax.ShapeDtypeStruct((N_ROWS, 1), jnp.int32),
    scratch_shapes=[
        pltpu.VMEM((N_ROWS, 128), jnp.float32),
        pltpu.VMEM((N_ROWS, 128), jnp.int32),
    ],
)


def kernel(x):
    return _argmin(x)[:, 0]
